# baseline (speedup 1.0000x reference)
import functools

import jax
import jax.numpy as jnp
from jax import lax
from jax.experimental import pallas as pl
from jax.experimental.pallas import tpu as pltpu
from jax.experimental.pallas import tpu_sc as plsc

B = 16384
V1, D1 = 10000, 128
V2, D2 = 5000, 64
OUT = 10
DP = 16
PACK = 8
R1ROWS = V1 // PACK
R2ROWS = V2 // PACK

NC = 2
NS = 16
NW = NC * NS
BPW = B // NW
IDX_ROW = 128
NCHUNK = BPW // IDX_ROW
TPAD = BPW + 1


def _aligned_chunks(total, step):
    out = []
    off = 0
    while off < total:
        out.append((off, min(step, total - off)))
        off += step
    return out


_T1_CHUNKS = _aligned_chunks(V1, 1248)
_T2_CHUNKS = _aligned_chunks(V2, 624)


def _proj_body(t1_hbm, t2_hbm, w_ref, b_ref, o1_ref, o2a_ref, o2b_ref,
               t1v, t2v, sem1, sem2):
    c1 = [pltpu.make_async_copy(t1_hbm.at[pl.ds(off, sz), :],
                                t1v.at[pl.ds(off, sz), :], sem1)
          for off, sz in _T1_CHUNKS]
    c2 = [pltpu.make_async_copy(t2_hbm.at[pl.ds(off, sz), :],
                                t2v.at[pl.ds(off, sz), :], sem2)
          for off, sz in _T2_CHUNKS]
    for c in c1:
        c.start()
    for c in c2:
        c.start()
    w = w_ref[...]
    zpad = jnp.zeros((D1 + 2 * D2, DP - OUT), jnp.float32)
    w16 = jnp.concatenate([w, zpad], axis=1)
    w1 = w16[:D1]
    w2a = w16[D1:D1 + D2]
    w2b = w16[D1 + D2:]
    b16 = jnp.concatenate(
        [b_ref[...], jnp.zeros((1, DP - OUT), jnp.float32)], axis=1)
    done1 = done2 = 0
    for j in range(PACK):
        lanes = pl.ds(j * DP, DP)
        need1 = min(len(c1), -(-((j + 1) * R1ROWS) // 1248))
        while done1 < need1:
            c1[done1].wait()
            done1 += 1
        o1_ref[:, lanes] = jnp.dot(
            t1v[pl.ds(j * R1ROWS, R1ROWS), :], w1,
            preferred_element_type=jnp.float32) + b16
        need2 = min(len(c2), -(-((j + 1) * R2ROWS) // 624))
        while done2 < need2:
            c2[done2].wait()
            done2 += 1
        t2j = t2v[pl.ds(j * R2ROWS, R2ROWS), :]
        o2a_ref[:, lanes] = jnp.dot(t2j, w2a,
                                    preferred_element_type=jnp.float32)
        o2b_ref[:, lanes] = jnp.dot(t2j, w2b,
                                    preferred_element_type=jnp.float32)


_sc_mesh = plsc.VectorSubcoreMesh(core_axis_name="c", subcore_axis_name="s")


@functools.partial(
    pl.kernel,
    mesh=_sc_mesh,
    compiler_params=pltpu.CompilerParams(
        use_tc_tiling_on_sc=False, needs_layout_passes=False),
    out_type=jax.ShapeDtypeStruct((NW, DP), jnp.float32),
    scratch_types=[pltpu.VMEM((1, DP), jnp.float32)],
)
def _warmup(out, rt):
    wid = lax.axis_index("s") * NC + lax.axis_index("c")
    pltpu.sync_copy(rt, out.at[pl.ds(wid, 1)])


@functools.partial(
    pl.kernel,
    mesh=_sc_mesh,
    compiler_params=pltpu.CompilerParams(
        use_tc_tiling_on_sc=False, needs_layout_passes=False),
    out_type=jax.ShapeDtypeStruct((OUT, B), jnp.float32),
    scratch_types=[
        pltpu.VMEM((NCHUNK, IDX_ROW), jnp.int32),
        pltpu.VMEM((NCHUNK, IDX_ROW), jnp.int32),
        pltpu.VMEM((NCHUNK, IDX_ROW), jnp.int32),
        pltpu.VMEM((BPW, DP), jnp.float32),
        pltpu.VMEM((BPW, DP), jnp.float32),
        pltpu.VMEM((BPW, DP), jnp.float32),
        pltpu.VMEM((DP, TPAD), jnp.float32),
        pltpu.SemaphoreType.DMA,
        pltpu.SemaphoreType.DMA,
        pltpu.SemaphoreType.DMA,
        pltpu.SemaphoreType.DMA,
    ],
)
def _gather_sum(p1, p2a, p2b, i1, i2a, i2b, warm, out_t,
                idx1, idx2, idx3, r1, r2, r3, rt,
                sem0, sem1, sem2, sem3):
    del warm
    sems = [sem0, sem1, sem2, sem3]
    wid = lax.axis_index("s") * NC + lax.axis_index("c")
    rowbase = wid * NCHUNK
    ic = [pltpu.async_copy(i1.at[pl.ds(rowbase, NCHUNK)], idx1, sem0),
          pltpu.async_copy(i2a.at[pl.ds(rowbase, NCHUNK)], idx2, sem1),
          pltpu.async_copy(i2b.at[pl.ds(rowbase, NCHUNK)], idx3, sem2)]
    for c in ic:
        c.wait()
    handles = []
    for j in range(NCHUNK):
        dst = pl.ds(j * IDX_ROW, IDX_ROW)
        handles.append([
            pltpu.async_copy(p1.at[idx1.at[j]], r1.at[dst], sems[j]),
            pltpu.async_copy(p2a.at[idx2.at[j]], r2.at[dst], sems[j]),
            pltpu.async_copy(p2b.at[idx3.at[j]], r3.at[dst], sems[j]),
        ])

    lane = lax.iota(jnp.int32, DP)
    UNROLL = 4

    def body(iu, carry):
        for k in range(UNROLL):
            i = iu * UNROLL + k
            s = r1[i] + r2[i] + r3[i]
            plsc.store_scatter(rt, [lane, jnp.full((DP,), i, jnp.int32)], s)
        return carry

    step = IDX_ROW // UNROLL
    for j in range(NCHUNK):
        for c in handles[j]:
            c.wait()
        lax.fori_loop(j * step, (j + 1) * step, body, 0)

    pltpu.sync_copy(rt.at[pl.ds(0, OUT), pl.ds(0, BPW)],
                    out_t.at[:, pl.ds(wid * BPW, BPW)])


def _remap(v, nrows):
    return PACK * (v % nrows) + v // nrows


def kernel(indices1, indices2, table1, table2, W, b):
    W = W.astype(jnp.float32)

    p1p, p2ap, p2bp = pl.pallas_call(
        _proj_body,
        in_specs=[
            pl.BlockSpec(memory_space=pl.ANY),
            pl.BlockSpec(memory_space=pl.ANY),
            pl.BlockSpec(memory_space=pltpu.VMEM),
            pl.BlockSpec(memory_space=pltpu.VMEM),
        ],
        out_shape=(
            jax.ShapeDtypeStruct((R1ROWS, PACK * DP), jnp.float32),
            jax.ShapeDtypeStruct((R2ROWS, PACK * DP), jnp.float32),
            jax.ShapeDtypeStruct((R2ROWS, PACK * DP), jnp.float32),
        ),
        scratch_shapes=[
            pltpu.VMEM((V1, D1), jnp.float32),
            pltpu.VMEM((V2, D2), jnp.float32),
            pltpu.SemaphoreType.DMA,
            pltpu.SemaphoreType.DMA,
        ],
    )(table1, table2, W, b.reshape(1, OUT))

    p1 = p1p.reshape(V1, DP)
    p2a = p2ap.reshape(V2, DP)
    p2b = p2bp.reshape(V2, DP)

    i1 = _remap(indices1.astype(jnp.int32), R1ROWS).reshape(B // IDX_ROW, IDX_ROW)
    i2 = indices2.astype(jnp.int32)
    i2a = _remap(i2[:, 0], R2ROWS).reshape(B // IDX_ROW, IDX_ROW)
    i2b = _remap(i2[:, 1], R2ROWS).reshape(B // IDX_ROW, IDX_ROW)

    warm = _warmup()
    out_t = _gather_sum(p1, p2a, p2b, i1, i2a, i2b, warm)
    return out_t.T


# speedup vs baseline: 1.0191x; 1.0191x over previous
import functools

import jax
import jax.numpy as jnp
from jax import lax
from jax.experimental import pallas as pl
from jax.experimental.pallas import tpu as pltpu
from jax.experimental.pallas import tpu_sc as plsc

B = 16384
V1, D1 = 10000, 128
V2, D2 = 5000, 64
OUT = 10
DP = 16
PACK = 8
R1ROWS = V1 // PACK
R2ROWS = V2 // PACK

NC = 2
NS = 16
NW = NC * NS
BPW = B // NW
IDX_ROW = 128
NCHUNK = BPW // IDX_ROW
TPAD = BPW + 1


def _proj_body(t1_ref, t2_ref, w_ref, b_ref, o1_ref, o2a_ref, o2b_ref):
    w = w_ref[...]
    zpad = jnp.zeros((D1 + 2 * D2, DP - OUT), jnp.float32)
    w16 = jnp.concatenate([w, zpad], axis=1)
    w1 = w16[:D1]
    w2a = w16[D1:D1 + D2]
    w2b = w16[D1 + D2:]
    b16 = jnp.concatenate(
        [b_ref[...], jnp.zeros((1, DP - OUT), jnp.float32)], axis=1)
    for j in range(PACK):
        lanes = pl.ds(j * DP, DP)
        o1_ref[:, lanes] = jnp.dot(
            t1_ref[pl.ds(j * R1ROWS, R1ROWS), :], w1,
            preferred_element_type=jnp.float32) + b16
        t2j = t2_ref[pl.ds(j * R2ROWS, R2ROWS), :]
        o2a_ref[:, lanes] = jnp.dot(t2j, w2a,
                                    preferred_element_type=jnp.float32)
        o2b_ref[:, lanes] = jnp.dot(t2j, w2b,
                                    preferred_element_type=jnp.float32)


_sc_mesh = plsc.VectorSubcoreMesh(core_axis_name="c", subcore_axis_name="s")


@functools.partial(
    pl.kernel,
    mesh=_sc_mesh,
    compiler_params=pltpu.CompilerParams(
        use_tc_tiling_on_sc=False, needs_layout_passes=False),
    out_type=jax.ShapeDtypeStruct((NW, DP), jnp.float32),
    scratch_types=[pltpu.VMEM((1, DP), jnp.float32)],
)
def _warmup(out, rt):
    wid = lax.axis_index("s") * NC + lax.axis_index("c")
    pltpu.sync_copy(rt, out.at[pl.ds(wid, 1)])


@functools.partial(
    pl.kernel,
    mesh=_sc_mesh,
    compiler_params=pltpu.CompilerParams(
        use_tc_tiling_on_sc=False, needs_layout_passes=False),
    out_type=jax.ShapeDtypeStruct((OUT, B), jnp.float32),
    scratch_types=[
        pltpu.VMEM((NCHUNK, IDX_ROW), jnp.int32),
        pltpu.VMEM((NCHUNK, IDX_ROW), jnp.int32),
        pltpu.VMEM((NCHUNK, IDX_ROW), jnp.int32),
        pltpu.VMEM((BPW, DP), jnp.float32),
        pltpu.VMEM((BPW, DP), jnp.float32),
        pltpu.VMEM((BPW, DP), jnp.float32),
        pltpu.VMEM((DP, TPAD), jnp.float32),
        pltpu.SemaphoreType.DMA,
        pltpu.SemaphoreType.DMA,
        pltpu.SemaphoreType.DMA,
        pltpu.SemaphoreType.DMA,
    ],
)
def _gather_sum(p1, p2a, p2b, i1, i2a, i2b, warm, out_t,
                idx1, idx2, idx3, r1, r2, r3, rt,
                sem0, sem1, sem2, sem3):
    del warm
    sems = [sem0, sem1, sem2, sem3]
    wid = lax.axis_index("s") * NC + lax.axis_index("c")
    rowbase = wid * NCHUNK
    ic = [pltpu.async_copy(i1.at[pl.ds(rowbase, NCHUNK)], idx1, sem0),
          pltpu.async_copy(i2a.at[pl.ds(rowbase, NCHUNK)], idx2, sem1),
          pltpu.async_copy(i2b.at[pl.ds(rowbase, NCHUNK)], idx3, sem2)]
    for c in ic:
        c.wait()
    GROUPS = IDX_ROW // DP

    def _remap_loop(buf, nrows):
        def rbody(k, carry):
            g = k // GROUPS
            sl = pl.ds((k % GROUPS) * DP, DP)
            v = buf[g, sl]
            q = v // nrows
            buf[g, sl] = PACK * (v - q * nrows) + q
            return carry
        lax.fori_loop(0, NCHUNK * GROUPS, rbody, 0)

    _remap_loop(idx1, R1ROWS)
    _remap_loop(idx2, R2ROWS)
    _remap_loop(idx3, R2ROWS)
    handles = []
    for j in range(NCHUNK):
        dst = pl.ds(j * IDX_ROW, IDX_ROW)
        handles.append([
            pltpu.async_copy(p1.at[idx1.at[j]], r1.at[dst], sems[j]),
            pltpu.async_copy(p2a.at[idx2.at[j]], r2.at[dst], sems[j]),
            pltpu.async_copy(p2b.at[idx3.at[j]], r3.at[dst], sems[j]),
        ])

    lane = lax.iota(jnp.int32, DP)
    UNROLL = 4

    def body(iu, carry):
        for k in range(UNROLL):
            i = iu * UNROLL + k
            s = r1[i] + r2[i] + r3[i]
            plsc.store_scatter(rt, [lane, jnp.full((DP,), i, jnp.int32)], s)
        return carry

    step = IDX_ROW // UNROLL
    for j in range(NCHUNK):
        for c in handles[j]:
            c.wait()
        lax.fori_loop(j * step, (j + 1) * step, body, 0)

    pltpu.sync_copy(rt.at[pl.ds(0, OUT), pl.ds(0, BPW)],
                    out_t.at[:, pl.ds(wid * BPW, BPW)])


def kernel(indices1, indices2, table1, table2, W, b):
    W = W.astype(jnp.float32)

    p1p, p2ap, p2bp = pl.pallas_call(
        _proj_body,
        out_shape=(
            jax.ShapeDtypeStruct((R1ROWS, PACK * DP), jnp.float32),
            jax.ShapeDtypeStruct((R2ROWS, PACK * DP), jnp.float32),
            jax.ShapeDtypeStruct((R2ROWS, PACK * DP), jnp.float32),
        ),
    )(table1, table2, W, b.reshape(1, OUT))

    p1 = p1p.reshape(V1, DP)
    p2a = p2ap.reshape(V2, DP)
    p2b = p2bp.reshape(V2, DP)

    i1 = indices1.astype(jnp.int32).reshape(B // IDX_ROW, IDX_ROW)
    i2 = indices2.astype(jnp.int32)
    i2a = i2[:, 0].reshape(B // IDX_ROW, IDX_ROW)
    i2b = i2[:, 1].reshape(B // IDX_ROW, IDX_ROW)

    warm = _warmup()
    out_t = _gather_sum(p1, p2a, p2b, i1, i2a, i2b, warm)
    return out_t.T


# speedup vs baseline: 1.1363x; 1.1150x over previous
import functools

import jax
import jax.numpy as jnp
from jax import lax
from jax.experimental import pallas as pl
from jax.experimental.pallas import tpu as pltpu
from jax.experimental.pallas import tpu_sc as plsc

B = 16384
V1, D1 = 10000, 128
V2, D2 = 5000, 64
OUT = 10
DP = 16
PACK = 8
R1ROWS = V1 // PACK
R2ROWS = V2 // PACK

NC = 2
NS = 16
NW = NC * NS
BPW = B // NW
IDX_ROW = 128
NCHUNK = BPW // IDX_ROW
TPAD = BPW + 1


def _proj_body(t1_ref, t2_ref, w_ref, b_ref, o1_ref, o2a_ref, o2b_ref):
    w = w_ref[...]
    zpad = jnp.zeros((D1 + 2 * D2, DP - OUT), jnp.float32)
    w16 = jnp.concatenate([w, zpad], axis=1)
    w1 = w16[:D1]
    w2a = w16[D1:D1 + D2]
    w2b = w16[D1 + D2:]
    b16 = jnp.concatenate(
        [b_ref[...], jnp.zeros((1, DP - OUT), jnp.float32)], axis=1)
    for j in range(PACK):
        lanes = pl.ds(j * DP, DP)
        o1_ref[:, lanes] = jnp.dot(
            t1_ref[pl.ds(j * R1ROWS, R1ROWS), :], w1,
            preferred_element_type=jnp.float32) + b16
        t2j = t2_ref[pl.ds(j * R2ROWS, R2ROWS), :]
        o2a_ref[:, lanes] = jnp.dot(t2j, w2a,
                                    preferred_element_type=jnp.float32)
        o2b_ref[:, lanes] = jnp.dot(t2j, w2b,
                                    preferred_element_type=jnp.float32)


_sc_mesh = plsc.VectorSubcoreMesh(core_axis_name="c", subcore_axis_name="s")


@functools.partial(
    pl.kernel,
    mesh=_sc_mesh,
    compiler_params=pltpu.CompilerParams(
        use_tc_tiling_on_sc=False, needs_layout_passes=False),
    out_type=jax.ShapeDtypeStruct((NW, DP), jnp.float32),
    scratch_types=[pltpu.VMEM((1, DP), jnp.float32)],
)
def _warmup(out, rt):
    wid = lax.axis_index("s") * NC + lax.axis_index("c")
    pltpu.sync_copy(rt, out.at[pl.ds(wid, 1)])


@functools.partial(
    pl.kernel,
    mesh=_sc_mesh,
    compiler_params=pltpu.CompilerParams(
        use_tc_tiling_on_sc=False, needs_layout_passes=False),
    out_type=jax.ShapeDtypeStruct((OUT, B), jnp.float32),
    scratch_types=[
        pltpu.VMEM((NCHUNK, IDX_ROW), jnp.int32),
        pltpu.VMEM((NCHUNK, IDX_ROW), jnp.int32),
        pltpu.VMEM((NCHUNK, IDX_ROW), jnp.int32),
        pltpu.VMEM((BPW, DP), jnp.float32),
        pltpu.VMEM((BPW, DP), jnp.float32),
        pltpu.VMEM((BPW, DP), jnp.float32),
        pltpu.VMEM((DP, TPAD), jnp.float32),
        pltpu.SemaphoreType.DMA,
        pltpu.SemaphoreType.DMA,
        pltpu.SemaphoreType.DMA,
        pltpu.SemaphoreType.DMA,
    ],
)
def _gather_sum(p1, p2a, p2b, i_all, warm, out_t,
                idx1, idx2, idx3, r1, r2, r3, rt,
                sem0, sem1, sem2, sem3):
    del warm
    sems = [sem0, sem1, sem2, sem3]
    wid = lax.axis_index("s") * NC + lax.axis_index("c")
    rowbase = wid * NCHUNK
    nrow = B // IDX_ROW
    ic = [pltpu.async_copy(i_all.at[pl.ds(rowbase, NCHUNK)], idx1, sem0),
          pltpu.async_copy(i_all.at[pl.ds(nrow + rowbase, NCHUNK)], idx2, sem1),
          pltpu.async_copy(i_all.at[pl.ds(2 * nrow + rowbase, NCHUNK)], idx3, sem2)]
    for c in ic:
        c.wait()
    handles = []
    for j in range(NCHUNK):
        dst = pl.ds(j * IDX_ROW, IDX_ROW)
        handles.append([
            pltpu.async_copy(p1.at[idx1.at[j]], r1.at[dst], sems[j]),
            pltpu.async_copy(p2a.at[idx2.at[j]], r2.at[dst], sems[j]),
            pltpu.async_copy(p2b.at[idx3.at[j]], r3.at[dst], sems[j]),
        ])

    lane = lax.iota(jnp.int32, DP)
    UNROLL = 4

    def body(iu, carry):
        for k in range(UNROLL):
            i = iu * UNROLL + k
            s = r1[i] + r2[i] + r3[i]
            plsc.store_scatter(rt, [lane, jnp.full((DP,), i, jnp.int32)], s)
        return carry

    step = IDX_ROW // UNROLL
    for j in range(NCHUNK):
        for c in handles[j]:
            c.wait()
        lax.fori_loop(j * step, (j + 1) * step, body, 0)

    pltpu.sync_copy(rt.at[pl.ds(0, OUT), pl.ds(0, BPW)],
                    out_t.at[:, pl.ds(wid * BPW, BPW)])


def _remap(v, nrows):
    q = lax.div(v, jnp.int32(nrows))
    return PACK * (v - q * nrows) + q


def kernel(indices1, indices2, table1, table2, W, b):
    W = W.astype(jnp.float32)

    p1p, p2ap, p2bp = pl.pallas_call(
        _proj_body,
        out_shape=(
            jax.ShapeDtypeStruct((R1ROWS, PACK * DP), jnp.float32),
            jax.ShapeDtypeStruct((R2ROWS, PACK * DP), jnp.float32),
            jax.ShapeDtypeStruct((R2ROWS, PACK * DP), jnp.float32),
        ),
    )(table1, table2, W, b.reshape(1, OUT))

    p1 = p1p.reshape(V1, DP)
    p2a = p2ap.reshape(V2, DP)
    p2b = p2bp.reshape(V2, DP)

    i2 = indices2.astype(jnp.int32)
    i_all = jnp.stack([
        _remap(indices1.astype(jnp.int32), R1ROWS),
        _remap(i2[:, 0], R2ROWS),
        _remap(i2[:, 1], R2ROWS),
    ]).reshape(3 * (B // IDX_ROW), IDX_ROW)

    warm = _warmup()
    out_t = _gather_sum(p1, p2a, p2b, i_all, warm)
    return out_t.T
